# baseline (speedup 1.0000x reference)
import functools

import jax
import jax.numpy as jnp
from jax import lax
from jax.experimental import pallas as pl
from jax.experimental.pallas import tpu as pltpu
from jax.experimental.pallas import tpu_sc as plsc

NC, NS = 2, 16
NW = NC * NS
NSPLIT = 2


def _tc_preproject(h, W1e):
    N, D = h.shape
    H = W1e.shape[1]

    def body(h_ref, w1_ref, hs_ref, ht_ref):
        w1 = w1_ref[...]
        hv = h_ref[...]
        hs_ref[...] = jnp.dot(hv, w1[0:D], preferred_element_type=jnp.float32)
        ht_ref[...] = jnp.dot(hv, w1[D:2 * D], preferred_element_type=jnp.float32)

    return pl.pallas_call(
        body,
        grid=(1,),
        in_specs=[
            pl.BlockSpec((N, D), lambda i: (0, 0)),
            pl.BlockSpec(W1e.shape, lambda i: (0, 0)),
        ],
        out_specs=[
            pl.BlockSpec((N, H), lambda i: (0, 0)),
            pl.BlockSpec((N, H), lambda i: (0, 0)),
        ],
        out_shape=[jax.ShapeDtypeStruct((N, H), jnp.float32),
                   jax.ShapeDtypeStruct((N, H), jnp.float32)],
    )(h, W1e)


def _sc_gather(hs, ht, row, col):
    N, D2 = hs.shape
    E = row.shape[0]
    epw = E // NW
    CH = 200
    nchunk = epw // CH
    mesh = plsc.VectorSubcoreMesh(core_axis_name="c", subcore_axis_name="s")

    @functools.partial(
        pl.kernel,
        mesh=mesh,
        out_type=[jax.ShapeDtypeStruct((E, D2), jnp.int32),
                  jax.ShapeDtypeStruct((E, D2), jnp.int32)],
        scratch_types=[
            pltpu.VMEM((CH,), jnp.int32),
            pltpu.VMEM((CH,), jnp.int32),
            pltpu.VMEM((CH, D2), jnp.int32),
            pltpu.VMEM((CH, D2), jnp.int32),
            pltpu.SemaphoreType.DMA,
            pltpu.SemaphoreType.DMA,
        ],
        compiler_params=pltpu.CompilerParams(use_tc_tiling_on_sc=False),
    )
    def k(hs_hbm, ht_hbm, row_hbm, col_hbm, src_hbm, tgt_hbm,
          ri_v, ci_v, sr_v, tg_v, sem1, sem2):
        wid = lax.axis_index("s") * NC + lax.axis_index("c")
        base0 = wid * epw

        @pl.loop(0, nchunk)
        def _(i):
            base = base0 + i * CH
            pltpu.sync_copy(row_hbm.at[pl.ds(base, CH)], ri_v)
            pltpu.sync_copy(col_hbm.at[pl.ds(base, CH)], ci_v)
            a = pltpu.async_copy(hs_hbm.at[ri_v], sr_v, sem1)
            b = pltpu.async_copy(ht_hbm.at[ci_v], tg_v, sem2)
            a.wait()
            b.wait()
            pltpu.sync_copy(sr_v, src_hbm.at[pl.ds(base, CH)])
            pltpu.sync_copy(tg_v, tgt_hbm.at[pl.ds(base, CH)])

    return k(hs, ht, row, col)


def _sc_segment_sum(ef_halves, row, zeros):
    Eh, D = ef_halves[0].shape
    E = Eh * NC
    Npad = zeros.shape[0]
    SCH = 80
    epw = Eh // NS
    nchunk = epw // SCH
    rpt = Npad // NS
    mesh = plsc.VectorSubcoreMesh(core_axis_name="c", subcore_axis_name="s")

    @functools.partial(
        pl.kernel,
        mesh=mesh,
        out_type=jax.ShapeDtypeStruct((NC, Npad, D), jnp.float32),
        scratch_types=[
            pltpu.VMEM((SCH,), jnp.int32),
            pltpu.VMEM((SCH, D), jnp.float32),
            pltpu.VMEM_SHARED((Npad, D), jnp.float32),
        ],
    )
    def k(ef0_hbm, ef1_hbm, row_hbm, zero_hbm, out_hbm, idx_v, ef_v, acc_sh):
        c = lax.axis_index("c")
        s = lax.axis_index("s")
        zbase = s * rpt
        pltpu.sync_copy(zero_hbm.at[pl.ds(zbase, rpt)],
                        acc_sh.at[pl.ds(zbase, rpt)])
        plsc.subcore_barrier()

        for cc, ef_hbm in enumerate((ef0_hbm, ef1_hbm)):
            @pl.when(c == cc)
            def _():
                base0 = s * epw

                @pl.loop(0, nchunk)
                def _(i):
                    base = base0 + i * SCH
                    pltpu.sync_copy(row_hbm.at[pl.ds(cc * Eh + base, SCH)],
                                    idx_v)
                    pltpu.sync_copy(ef_hbm.at[pl.ds(base, SCH)], ef_v)
                    pltpu.sync_copy(ef_v, acc_sh.at[idx_v], add=True)

        plsc.subcore_barrier()
        pltpu.sync_copy(acc_sh.at[pl.ds(zbase, rpt)],
                        out_hbm.at[c, pl.ds(zbase, rpt)])

    return k(ef_halves[0], ef_halves[1], row, zeros)


def _unpack_bf16_pair(w):
    u = lax.bitcast_convert_type(w, jnp.uint32)
    lo = lax.bitcast_convert_type((u & 0xFFFF).astype(jnp.uint16),
                                  jnp.bfloat16).astype(jnp.float32)
    hi = lax.bitcast_convert_type((u >> 16).astype(jnp.uint16),
                                  jnp.bfloat16).astype(jnp.float32)
    return jnp.concatenate([lo, hi], axis=1)


def _tc_edge_mlp(src, tgt, attr_t, W1e, b1e, W2e, b2e):
    E, D2 = src.shape
    DE = attr_t.shape[0]
    H = W2e.shape[0]
    BE = 6400

    def body(src_ref, tgt_ref, attr_ref, w1c_ref, b1_ref, w2_ref, b2_ref, out_ref):
        za = lax.dot_general(attr_ref[...], w1c_ref[...],
                             (((0,), (0,)), ((), ())),
                             preferred_element_type=jnp.float32)
        z = (_unpack_bf16_pair(src_ref[...]) + _unpack_bf16_pair(tgt_ref[...])
             + za + b1_ref[...])
        m = z * jax.nn.sigmoid(z)
        y = jnp.dot(m, w2_ref[...],
                    preferred_element_type=jnp.float32) + b2_ref[...]
        out_ref[...] = y * jax.nn.sigmoid(y)

    return pl.pallas_call(
        body,
        grid=(E // BE,),
        in_specs=[
            pl.BlockSpec((BE, D2), lambda i: (i, 0)),
            pl.BlockSpec((BE, D2), lambda i: (i, 0)),
            pl.BlockSpec((DE, BE), lambda i: (0, i)),
            pl.BlockSpec((DE, H), lambda i: (0, 0)),
            pl.BlockSpec((1, H), lambda i: (0, 0)),
            pl.BlockSpec((H, H), lambda i: (0, 0)),
            pl.BlockSpec((1, H), lambda i: (0, 0)),
        ],
        out_specs=pl.BlockSpec((BE, H), lambda i: (i, 0)),
        out_shape=jax.ShapeDtypeStruct((E, H), jnp.float32),
    )(src, tgt, attr_t, W1e[4 * D2:4 * D2 + DE], b1e.reshape(1, H), W2e,
      b2e.reshape(1, H))


def _tc_node_mlp(h, p0, p1, W1n, b1n, W2n, b2n):
    N, D = h.shape
    H = p0.shape[1]
    DO = W2n.shape[1]
    BN = 2000

    def body(h_ref, p0_ref, p1_ref, w1_ref, b1_ref, w2_ref, b2_ref, out_ref):
        agg = p0_ref[...] + p1_ref[...]
        w1 = w1_ref[...]
        z = (jnp.dot(h_ref[...], w1[0:D], preferred_element_type=jnp.float32)
             + jnp.dot(agg, w1[D:D + H], preferred_element_type=jnp.float32)
             + b1_ref[...])
        t = z * jax.nn.sigmoid(z)
        out_ref[...] = jnp.dot(t, w2_ref[...],
                               preferred_element_type=jnp.float32) + b2_ref[...]

    return pl.pallas_call(
        body,
        grid=(N // BN,),
        in_specs=[
            pl.BlockSpec((BN, D), lambda i: (i, 0)),
            pl.BlockSpec((BN, H), lambda i: (i, 0)),
            pl.BlockSpec((BN, H), lambda i: (i, 0)),
            pl.BlockSpec((D + H, H), lambda i: (0, 0)),
            pl.BlockSpec((1, H), lambda i: (0, 0)),
            pl.BlockSpec((H, DO), lambda i: (0, 0)),
            pl.BlockSpec((1, DO), lambda i: (0, 0)),
        ],
        out_specs=pl.BlockSpec((BN, DO), lambda i: (i, 0)),
        out_shape=jax.ShapeDtypeStruct((N, DO), jnp.float32),
    )(h, p0, p1, W1n, b1n.reshape(1, H), W2n, b2n.reshape(1, DO))


def kernel(h, edge_index, edge_attr, W1e, b1e, W2e, b2e, W1n, b1n, W2n, b2n):
    N = h.shape[0]
    E = edge_index.shape[1]
    Eh = E // NSPLIT
    row = edge_index[0]
    col = edge_index[1]
    attr_t = edge_attr.T
    hs, ht = _tc_preproject(h, W1e)

    def pack(x):
        xb = x.astype(jnp.bfloat16)
        D = x.shape[1]
        lo = lax.bitcast_convert_type(xb[:, :D // 2], jnp.uint16).astype(jnp.uint32)
        hi = lax.bitcast_convert_type(xb[:, D // 2:], jnp.uint16).astype(jnp.uint32)
        return lax.bitcast_convert_type(lo | (hi << 16), jnp.int32)

    hs, ht = pack(hs), pack(ht)
    ef_halves = []
    for ci in range(NSPLIT):
        lo = ci * Eh
        src, tgt = _sc_gather(hs, ht, row[lo:lo + Eh], col[lo:lo + Eh])
        ef_halves.append(_tc_edge_mlp(src, tgt, attr_t[:, lo:lo + Eh],
                                      W1e, b1e, W2e, b2e))
    Npad = ((N + 8 * NS - 1) // (8 * NS)) * (8 * NS)
    zeros = jnp.zeros((Npad, h.shape[1]), jnp.float32)
    p = _sc_segment_sum(ef_halves, row, zeros)
    return _tc_node_mlp(h, p[0, :N], p[1, :N], W1n, b1n, W2n, b2n)


# speedup vs baseline: 1.4089x; 1.4089x over previous
import functools

import jax
import jax.numpy as jnp
from jax import lax
from jax.experimental import pallas as pl
from jax.experimental.pallas import tpu as pltpu
from jax.experimental.pallas import tpu_sc as plsc

NC, NS = 2, 16
NW = NC * NS
NSPLIT = 2


def _tc_preproject(h, W1e):
    N, D = h.shape
    H = W1e.shape[1]

    def body(h_ref, w1_ref, hs_ref, ht_ref):
        w1 = w1_ref[...]
        hv = h_ref[...]
        hs_ref[...] = jnp.dot(hv, w1[0:D], preferred_element_type=jnp.float32)
        ht_ref[...] = jnp.dot(hv, w1[D:2 * D], preferred_element_type=jnp.float32)

    return pl.pallas_call(
        body,
        grid=(1,),
        in_specs=[
            pl.BlockSpec((N, D), lambda i: (0, 0)),
            pl.BlockSpec(W1e.shape, lambda i: (0, 0)),
        ],
        out_specs=[
            pl.BlockSpec((N, H), lambda i: (0, 0)),
            pl.BlockSpec((N, H), lambda i: (0, 0)),
        ],
        out_shape=[jax.ShapeDtypeStruct((N, H), jnp.float32),
                   jax.ShapeDtypeStruct((N, H), jnp.float32)],
    )(h, W1e)


def _sc_gather(hs, ht, row, col):
    N, D2 = hs.shape
    E = row.shape[0]
    epw = E // NW
    CH = 200
    nchunk = epw // CH
    mesh = plsc.VectorSubcoreMesh(core_axis_name="c", subcore_axis_name="s")

    @functools.partial(
        pl.kernel,
        mesh=mesh,
        out_type=jax.ShapeDtypeStruct((E, 2 * D2), jnp.int32),
        scratch_types=[
            pltpu.VMEM((CH,), jnp.int32),
            pltpu.VMEM((CH,), jnp.int32),
            pltpu.VMEM((CH, D2), jnp.int32),
            pltpu.VMEM((CH, D2), jnp.int32),
            pltpu.SemaphoreType.DMA,
            pltpu.SemaphoreType.DMA,
        ],
        compiler_params=pltpu.CompilerParams(use_tc_tiling_on_sc=False),
    )
    def k(hs_hbm, ht_hbm, row_hbm, col_hbm, st_hbm,
          ri_v, ci_v, sr_v, tg_v, sem1, sem2):
        wid = lax.axis_index("s") * NC + lax.axis_index("c")
        base0 = wid * epw

        @pl.loop(0, nchunk)
        def _(i):
            base = base0 + i * CH
            pltpu.sync_copy(row_hbm.at[pl.ds(base, CH)], ri_v)
            pltpu.sync_copy(col_hbm.at[pl.ds(base, CH)], ci_v)
            a = pltpu.async_copy(hs_hbm.at[ri_v], sr_v, sem1)
            b = pltpu.async_copy(ht_hbm.at[ci_v], tg_v, sem2)
            a.wait()
            b.wait()
            pltpu.sync_copy(sr_v, st_hbm.at[pl.ds(base, CH), pl.ds(0, D2)])
            pltpu.sync_copy(tg_v, st_hbm.at[pl.ds(base, CH), pl.ds(D2, D2)])

    return k(hs, ht, row, col)


def _sc_segment_sum(ef_halves, row, zeros):
    Eh, D = ef_halves[0].shape
    E = Eh * NC
    Npad = zeros.shape[0]
    SCH = 80
    epw = Eh // NS
    nchunk = epw // SCH
    rpt = Npad // NS
    mesh = plsc.VectorSubcoreMesh(core_axis_name="c", subcore_axis_name="s")

    @functools.partial(
        pl.kernel,
        mesh=mesh,
        out_type=jax.ShapeDtypeStruct((NC, Npad, D), jnp.float32),
        scratch_types=[
            pltpu.VMEM((SCH,), jnp.int32),
            pltpu.VMEM((SCH, D), jnp.float32),
            pltpu.VMEM_SHARED((Npad, D), jnp.float32),
        ],
    )
    def k(ef0_hbm, ef1_hbm, row_hbm, zero_hbm, out_hbm, idx_v, ef_v, acc_sh):
        c = lax.axis_index("c")
        s = lax.axis_index("s")
        zbase = s * rpt
        pltpu.sync_copy(zero_hbm.at[pl.ds(zbase, rpt)],
                        acc_sh.at[pl.ds(zbase, rpt)])
        plsc.subcore_barrier()

        for cc, ef_hbm in enumerate((ef0_hbm, ef1_hbm)):
            @pl.when(c == cc)
            def _():
                base0 = s * epw

                @pl.loop(0, nchunk)
                def _(i):
                    base = base0 + i * SCH
                    pltpu.sync_copy(row_hbm.at[pl.ds(cc * Eh + base, SCH)],
                                    idx_v)
                    pltpu.sync_copy(ef_hbm.at[pl.ds(base, SCH)], ef_v)
                    pltpu.sync_copy(ef_v, acc_sh.at[idx_v], add=True)

        plsc.subcore_barrier()
        pltpu.sync_copy(acc_sh.at[pl.ds(zbase, rpt)],
                        out_hbm.at[c, pl.ds(zbase, rpt)])

    return k(ef_halves[0], ef_halves[1], row, zeros)


def _unpack_bf16_pair(w):
    u = lax.bitcast_convert_type(w, jnp.uint32)
    lo = lax.bitcast_convert_type((u & 0xFFFF).astype(jnp.uint16),
                                  jnp.bfloat16).astype(jnp.float32)
    hi = lax.bitcast_convert_type((u >> 16).astype(jnp.uint16),
                                  jnp.bfloat16).astype(jnp.float32)
    return jnp.concatenate([lo, hi], axis=1)


def _tc_edge_mlp(st, attr_t, W1e, b1e, W2e, b2e):
    E, D = st.shape
    D2 = D // 2
    DE = attr_t.shape[0]
    H = W2e.shape[0]
    BE = 6400

    def body(st_ref, attr_ref, w1c_ref, b1_ref, w2_ref, b2_ref, out_ref):
        za = lax.dot_general(attr_ref[...], w1c_ref[...],
                             (((0,), (0,)), ((), ())),
                             preferred_element_type=jnp.float32)
        w = st_ref[...]
        z = (_unpack_bf16_pair(w[:, :D2]) + _unpack_bf16_pair(w[:, D2:])
             + za + b1_ref[...])
        m = z * jax.nn.sigmoid(z)
        y = jnp.dot(m, w2_ref[...],
                    preferred_element_type=jnp.float32) + b2_ref[...]
        out_ref[...] = y * jax.nn.sigmoid(y)

    return pl.pallas_call(
        body,
        grid=(E // BE,),
        in_specs=[
            pl.BlockSpec((BE, D), lambda i: (i, 0)),
            pl.BlockSpec((DE, BE), lambda i: (0, i)),
            pl.BlockSpec((DE, H), lambda i: (0, 0)),
            pl.BlockSpec((1, H), lambda i: (0, 0)),
            pl.BlockSpec((H, H), lambda i: (0, 0)),
            pl.BlockSpec((1, H), lambda i: (0, 0)),
        ],
        out_specs=pl.BlockSpec((BE, H), lambda i: (i, 0)),
        out_shape=jax.ShapeDtypeStruct((E, H), jnp.float32),
    )(st, attr_t, W1e[2 * D:2 * D + DE], b1e.reshape(1, H), W2e,
      b2e.reshape(1, H))


def _tc_node_mlp(h, p0, p1, W1n, b1n, W2n, b2n):
    N, D = h.shape
    H = p0.shape[1]
    DO = W2n.shape[1]
    BN = 2000

    def body(h_ref, p0_ref, p1_ref, w1_ref, b1_ref, w2_ref, b2_ref, out_ref):
        agg = p0_ref[...] + p1_ref[...]
        w1 = w1_ref[...]
        z = (jnp.dot(h_ref[...], w1[0:D], preferred_element_type=jnp.float32)
             + jnp.dot(agg, w1[D:D + H], preferred_element_type=jnp.float32)
             + b1_ref[...])
        t = z * jax.nn.sigmoid(z)
        out_ref[...] = jnp.dot(t, w2_ref[...],
                               preferred_element_type=jnp.float32) + b2_ref[...]

    return pl.pallas_call(
        body,
        grid=(N // BN,),
        in_specs=[
            pl.BlockSpec((BN, D), lambda i: (i, 0)),
            pl.BlockSpec((BN, H), lambda i: (i, 0)),
            pl.BlockSpec((BN, H), lambda i: (i, 0)),
            pl.BlockSpec((D + H, H), lambda i: (0, 0)),
            pl.BlockSpec((1, H), lambda i: (0, 0)),
            pl.BlockSpec((H, DO), lambda i: (0, 0)),
            pl.BlockSpec((1, DO), lambda i: (0, 0)),
        ],
        out_specs=pl.BlockSpec((BN, DO), lambda i: (i, 0)),
        out_shape=jax.ShapeDtypeStruct((N, DO), jnp.float32),
    )(h, p0, p1, W1n, b1n.reshape(1, H), W2n, b2n.reshape(1, DO))


def kernel(h, edge_index, edge_attr, W1e, b1e, W2e, b2e, W1n, b1n, W2n, b2n):
    N = h.shape[0]
    E = edge_index.shape[1]
    Eh = E // NSPLIT
    row = edge_index[0]
    col = edge_index[1]
    attr_t = edge_attr.T
    hs, ht = _tc_preproject(h, W1e)

    def pack(x):
        xb = x.astype(jnp.bfloat16)
        D = x.shape[1]
        lo = lax.bitcast_convert_type(xb[:, :D // 2], jnp.uint16).astype(jnp.uint32)
        hi = lax.bitcast_convert_type(xb[:, D // 2:], jnp.uint16).astype(jnp.uint32)
        return lax.bitcast_convert_type(lo | (hi << 16), jnp.int32)

    hs, ht = pack(hs), pack(ht)
    ef_halves = []
    for ci in range(NSPLIT):
        lo = ci * Eh
        st = _sc_gather(hs, ht, row[lo:lo + Eh], col[lo:lo + Eh])
        ef_halves.append(_tc_edge_mlp(st, attr_t[:, lo:lo + Eh],
                                      W1e, b1e, W2e, b2e))
    Npad = ((N + 8 * NS - 1) // (8 * NS)) * (8 * NS)
    zeros = jnp.zeros((Npad, h.shape[1]), jnp.float32)
    p = _sc_segment_sum(ef_halves, row, zeros)
    return _tc_node_mlp(h, p[0, :N], p[1, :N], W1n, b1n, W2n, b2n)


# speedup vs baseline: 1.8034x; 1.2800x over previous
import functools

import jax
import jax.numpy as jnp
from jax import lax
from jax.experimental import pallas as pl
from jax.experimental.pallas import tpu as pltpu
from jax.experimental.pallas import tpu_sc as plsc

NC, NS = 2, 16
NW = NC * NS
NSPLIT = 2


def _tc_preproject(h, W1e):
    N, D = h.shape
    H = W1e.shape[1]

    def body(h_ref, w1_ref, hs_ref, ht_ref):
        w1 = w1_ref[...]
        hv = h_ref[...]
        hs_ref[...] = jnp.dot(hv, w1[0:D], preferred_element_type=jnp.float32)
        ht_ref[...] = jnp.dot(hv, w1[D:2 * D], preferred_element_type=jnp.float32)

    return pl.pallas_call(
        body,
        grid=(1,),
        in_specs=[
            pl.BlockSpec((N, D), lambda i: (0, 0)),
            pl.BlockSpec(W1e.shape, lambda i: (0, 0)),
        ],
        out_specs=[
            pl.BlockSpec((N, H), lambda i: (0, 0)),
            pl.BlockSpec((N, H), lambda i: (0, 0)),
        ],
        out_shape=[jax.ShapeDtypeStruct((N, H), jnp.float32),
                   jax.ShapeDtypeStruct((N, H), jnp.float32)],
    )(h, W1e)


def _sc_gather(hs, ht, row, col):
    N, D2 = hs.shape
    E = row.shape[0]
    epw = E // NW
    CH = 200
    nchunk = epw // CH
    mesh = plsc.VectorSubcoreMesh(core_axis_name="c", subcore_axis_name="s")

    @functools.partial(
        pl.kernel,
        mesh=mesh,
        out_type=jax.ShapeDtypeStruct((E, 2 * D2), jnp.int32),
        scratch_types=[
            pltpu.VMEM((CH,), jnp.int32),
            pltpu.VMEM((CH,), jnp.int32),
            pltpu.VMEM((CH, D2), jnp.int32),
            pltpu.VMEM((CH, D2), jnp.int32),
            pltpu.SemaphoreType.DMA,
            pltpu.SemaphoreType.DMA,
        ],
        compiler_params=pltpu.CompilerParams(use_tc_tiling_on_sc=False),
    )
    def k(hs_hbm, ht_hbm, row_hbm, col_hbm, st_hbm,
          ri_v, ci_v, sr_v, tg_v, sem1, sem2):
        wid = lax.axis_index("s") * NC + lax.axis_index("c")
        base0 = wid * epw

        @pl.loop(0, nchunk)
        def _(i):
            base = base0 + i * CH
            pltpu.sync_copy(row_hbm.at[pl.ds(base, CH)], ri_v)
            pltpu.sync_copy(col_hbm.at[pl.ds(base, CH)], ci_v)
            a = pltpu.async_copy(hs_hbm.at[ri_v], sr_v, sem1)
            b = pltpu.async_copy(ht_hbm.at[ci_v], tg_v, sem2)
            a.wait()
            b.wait()
            pltpu.sync_copy(sr_v, st_hbm.at[pl.ds(base, CH), pl.ds(0, D2)])
            pltpu.sync_copy(tg_v, st_hbm.at[pl.ds(base, CH), pl.ds(D2, D2)])

    return k(hs, ht, row, col)


def _sc_segment_sum(ef, row, zeros):
    Eh, D = ef.shape
    Npad = zeros.shape[0]
    SCH = 40
    epc = Eh // NC
    epw = epc // NS
    nchunk = epw // SCH
    rpt = Npad // NS
    mesh = plsc.VectorSubcoreMesh(core_axis_name="c", subcore_axis_name="s")

    @functools.partial(
        pl.kernel,
        mesh=mesh,
        out_type=jax.ShapeDtypeStruct((NC, Npad, D), jnp.float32),
        scratch_types=[
            pltpu.VMEM((SCH,), jnp.int32),
            pltpu.VMEM((SCH,), jnp.int32),
            pltpu.VMEM((SCH, D), jnp.float32),
            pltpu.VMEM((SCH, D), jnp.float32),
            pltpu.VMEM_SHARED((Npad, D), jnp.float32),
            pltpu.SemaphoreType.DMA,
            pltpu.SemaphoreType.DMA,
            pltpu.SemaphoreType.DMA,
            pltpu.SemaphoreType.DMA,
        ],
    )
    def k(ef_hbm, row_hbm, zero_hbm, out_hbm, idx0, idx1, ef0, ef1, acc_sh,
          si0, si1, se0, se1):
        c = lax.axis_index("c")
        s = lax.axis_index("s")
        zbase = s * rpt
        pltpu.sync_copy(zero_hbm.at[pl.ds(zbase, rpt)],
                        acc_sh.at[pl.ds(zbase, rpt)])
        plsc.subcore_barrier()

        base0 = c * epc + s * epw

        def start(i, idxb, efb, sib, seb):
            base = base0 + i * SCH
            pltpu.make_async_copy(row_hbm.at[pl.ds(base, SCH)], idxb,
                                  sib).start()
            pltpu.make_async_copy(ef_hbm.at[pl.ds(base, SCH)], efb,
                                  seb).start()

        def flush(i, idxb, efb, sib, seb):
            base = base0 + i * SCH
            pltpu.make_async_copy(row_hbm.at[pl.ds(base, SCH)], idxb,
                                  sib).wait()
            pltpu.make_async_copy(ef_hbm.at[pl.ds(base, SCH)], efb,
                                  seb).wait()
            pltpu.sync_copy(efb, acc_sh.at[idxb], add=True)

        start(0, idx0, ef0, si0, se0)

        @pl.loop(0, (nchunk - 1) // 2)
        def _(j):
            i0 = 2 * j
            start(i0 + 1, idx1, ef1, si1, se1)
            flush(i0, idx0, ef0, si0, se0)
            start(i0 + 2, idx0, ef0, si0, se0)
            flush(i0 + 1, idx1, ef1, si1, se1)

        flush(nchunk - 1, idx0, ef0, si0, se0)

        plsc.subcore_barrier()
        pltpu.sync_copy(acc_sh.at[pl.ds(zbase, rpt)],
                        out_hbm.at[c, pl.ds(zbase, rpt)])

    return k(ef, row, zeros)


def _unpack_bf16_pair(w):
    u = lax.bitcast_convert_type(w, jnp.uint32)
    lo = lax.bitcast_convert_type((u & 0xFFFF).astype(jnp.uint16),
                                  jnp.bfloat16).astype(jnp.float32)
    hi = lax.bitcast_convert_type((u >> 16).astype(jnp.uint16),
                                  jnp.bfloat16).astype(jnp.float32)
    return jnp.concatenate([lo, hi], axis=1)


def _tc_edge_mlp(st, attr_t, W1e, b1e, W2e, b2e):
    E, D = st.shape
    D2 = D // 2
    DE = attr_t.shape[0]
    H = W2e.shape[0]
    BE = 6400

    def body(st_ref, attr_ref, w1c_ref, b1_ref, w2_ref, b2_ref, out_ref):
        za = lax.dot_general(attr_ref[...], w1c_ref[...],
                             (((0,), (0,)), ((), ())),
                             preferred_element_type=jnp.float32)
        w = st_ref[...]
        z = (_unpack_bf16_pair(w[:, :D2]) + _unpack_bf16_pair(w[:, D2:])
             + za + b1_ref[...])
        m = z * jax.nn.sigmoid(z)
        y = jnp.dot(m, w2_ref[...],
                    preferred_element_type=jnp.float32) + b2_ref[...]
        out_ref[...] = y * jax.nn.sigmoid(y)

    return pl.pallas_call(
        body,
        grid=(E // BE,),
        in_specs=[
            pl.BlockSpec((BE, D), lambda i: (i, 0)),
            pl.BlockSpec((DE, BE), lambda i: (0, i)),
            pl.BlockSpec((DE, H), lambda i: (0, 0)),
            pl.BlockSpec((1, H), lambda i: (0, 0)),
            pl.BlockSpec((H, H), lambda i: (0, 0)),
            pl.BlockSpec((1, H), lambda i: (0, 0)),
        ],
        out_specs=pl.BlockSpec((BE, H), lambda i: (i, 0)),
        out_shape=jax.ShapeDtypeStruct((E, H), jnp.float32),
    )(st, attr_t, W1e[2 * D:2 * D + DE], b1e.reshape(1, H), W2e,
      b2e.reshape(1, H))


def _tc_node_mlp(h, parts, W1n, b1n, W2n, b2n):
    N, D = h.shape
    H = parts[0].shape[1]
    DO = W2n.shape[1]
    BN = 2000

    def body(h_ref, p0_ref, p1_ref, p2_ref, p3_ref, w1_ref, b1_ref, w2_ref,
             b2_ref, out_ref):
        agg = ((p0_ref[...] + p1_ref[...]) + (p2_ref[...] + p3_ref[...]))
        w1 = w1_ref[...]
        z = (jnp.dot(h_ref[...], w1[0:D], preferred_element_type=jnp.float32)
             + jnp.dot(agg, w1[D:D + H], preferred_element_type=jnp.float32)
             + b1_ref[...])
        t = z * jax.nn.sigmoid(z)
        out_ref[...] = jnp.dot(t, w2_ref[...],
                               preferred_element_type=jnp.float32) + b2_ref[...]

    return pl.pallas_call(
        body,
        grid=(N // BN,),
        in_specs=[
            pl.BlockSpec((BN, D), lambda i: (i, 0)),
            pl.BlockSpec((BN, H), lambda i: (i, 0)),
            pl.BlockSpec((BN, H), lambda i: (i, 0)),
            pl.BlockSpec((BN, H), lambda i: (i, 0)),
            pl.BlockSpec((BN, H), lambda i: (i, 0)),
            pl.BlockSpec((D + H, H), lambda i: (0, 0)),
            pl.BlockSpec((1, H), lambda i: (0, 0)),
            pl.BlockSpec((H, DO), lambda i: (0, 0)),
            pl.BlockSpec((1, DO), lambda i: (0, 0)),
        ],
        out_specs=pl.BlockSpec((BN, DO), lambda i: (i, 0)),
        out_shape=jax.ShapeDtypeStruct((N, DO), jnp.float32),
    )(h, *parts, W1n, b1n.reshape(1, H), W2n, b2n.reshape(1, DO))


def kernel(h, edge_index, edge_attr, W1e, b1e, W2e, b2e, W1n, b1n, W2n, b2n):
    N = h.shape[0]
    E = edge_index.shape[1]
    Eh = E // NSPLIT
    row = edge_index[0]
    col = edge_index[1]
    attr_t = edge_attr.T
    hs, ht = _tc_preproject(h, W1e)

    def pack(x):
        xb = x.astype(jnp.bfloat16)
        D = x.shape[1]
        lo = lax.bitcast_convert_type(xb[:, :D // 2], jnp.uint16).astype(jnp.uint32)
        hi = lax.bitcast_convert_type(xb[:, D // 2:], jnp.uint16).astype(jnp.uint32)
        return lax.bitcast_convert_type(lo | (hi << 16), jnp.int32)

    hs, ht = pack(hs), pack(ht)
    Npad = ((N + 8 * NS - 1) // (8 * NS)) * (8 * NS)
    zeros = jnp.zeros((Npad, h.shape[1]), jnp.float32)
    parts = []
    for ci in range(NSPLIT):
        lo = ci * Eh
        st = _sc_gather(hs, ht, row[lo:lo + Eh], col[lo:lo + Eh])
        ef = _tc_edge_mlp(st, attr_t[:, lo:lo + Eh], W1e, b1e, W2e, b2e)
        p = _sc_segment_sum(ef, row[lo:lo + Eh], zeros)
        parts.extend([p[0, :N], p[1, :N]])
    return _tc_node_mlp(h, parts, W1n, b1n, W2n, b2n)


# speedup vs baseline: 2.0067x; 1.1127x over previous
import functools

import jax
import jax.numpy as jnp
from jax import lax
from jax.experimental import pallas as pl
from jax.experimental.pallas import tpu as pltpu
from jax.experimental.pallas import tpu_sc as plsc

NC, NS = 2, 16
NW = NC * NS
NSPLIT = 2


def _tc_preproject(h, W1e):
    N, D = h.shape
    H = W1e.shape[1]

    def body(h_ref, w1_ref, hs_ref, ht_ref):
        w1 = w1_ref[...]
        hv = h_ref[...]
        hs_ref[...] = jnp.dot(hv, w1[0:D], preferred_element_type=jnp.float32)
        ht_ref[...] = jnp.dot(hv, w1[D:2 * D], preferred_element_type=jnp.float32)

    return pl.pallas_call(
        body,
        grid=(1,),
        in_specs=[
            pl.BlockSpec((N, D), lambda i: (0, 0)),
            pl.BlockSpec(W1e.shape, lambda i: (0, 0)),
        ],
        out_specs=[
            pl.BlockSpec((N, H), lambda i: (0, 0)),
            pl.BlockSpec((N, H), lambda i: (0, 0)),
        ],
        out_shape=[jax.ShapeDtypeStruct((N, H), jnp.float32),
                   jax.ShapeDtypeStruct((N, H), jnp.float32)],
    )(h, W1e)


def _sc_gather(hs, ht, rowcol, E, off, Eh):
    N, D2 = hs.shape
    epw = Eh // NW
    CH = 200
    nchunk = epw // CH
    npair = nchunk // 2
    mesh = plsc.VectorSubcoreMesh(core_axis_name="c", subcore_axis_name="s")

    bufs = [pltpu.VMEM((CH,), jnp.int32), pltpu.VMEM((CH,), jnp.int32),
            pltpu.VMEM((CH, D2), jnp.int32), pltpu.VMEM((CH, D2), jnp.int32)]

    @functools.partial(
        pl.kernel,
        mesh=mesh,
        out_type=jax.ShapeDtypeStruct((Eh, 2 * D2), jnp.int32),
        scratch_types=bufs + bufs + [pltpu.SemaphoreType.DMA] * 12,
        compiler_params=pltpu.CompilerParams(use_tc_tiling_on_sc=False),
    )
    def k(hs_hbm, ht_hbm, ei_hbm, st_hbm,
          riA, ciA, srA, tgA, riB, ciB, srB, tgB, *sems):
        wid = lax.axis_index("s") * NC + lax.axis_index("c")
        base0 = wid * epw

        def idx_start(i, ri, ci, sms):
            base = off + base0 + i * CH
            return (pltpu.async_copy(ei_hbm.at[pl.ds(base, CH)], ri, sms[0]),
                    pltpu.async_copy(ei_hbm.at[pl.ds(E + base, CH)], ci,
                                     sms[1]))

        def gat_start(ri, ci, sr, tg, sms):
            return (pltpu.async_copy(hs_hbm.at[ri], sr, sms[2]),
                    pltpu.async_copy(ht_hbm.at[ci], tg, sms[3]))

        def out_start(i, sr, tg, sms):
            base = base0 + i * CH
            return (pltpu.async_copy(
                        sr, st_hbm.at[pl.ds(base, CH), pl.ds(0, D2)], sms[4]),
                    pltpu.async_copy(
                        tg, st_hbm.at[pl.ds(base, CH), pl.ds(D2, D2)], sms[5]))

        smA, smB = sems[:6], sems[6:]

        @pl.loop(0, npair)
        def _(j):
            i0 = 2 * j
            ia = idx_start(i0, riA, ciA, smA)
            ib = idx_start(i0 + 1, riB, ciB, smB)
            ia[0].wait()
            ia[1].wait()
            ga = gat_start(riA, ciA, srA, tgA, smA)
            ib[0].wait()
            ib[1].wait()
            gb = gat_start(riB, ciB, srB, tgB, smB)
            ga[0].wait()
            ga[1].wait()
            wa = out_start(i0, srA, tgA, smA)
            gb[0].wait()
            gb[1].wait()
            wb = out_start(i0 + 1, srB, tgB, smB)
            wa[0].wait()
            wa[1].wait()
            wb[0].wait()
            wb[1].wait()

        for i in range(2 * npair, nchunk):
            ia = idx_start(i, riA, ciA, smA)
            ia[0].wait()
            ia[1].wait()
            ga = gat_start(riA, ciA, srA, tgA, smA)
            ga[0].wait()
            ga[1].wait()
            wa = out_start(i, srA, tgA, smA)
            wa[0].wait()
            wa[1].wait()

    return k(hs, ht, rowcol)


def _sc_segment_sum(ef, rowcol, off, zeros):
    Eh, D = ef.shape
    Npad = zeros.shape[0]
    SCH = 40
    epc = Eh // NC
    epw = epc // NS
    nchunk = epw // SCH
    rpt = Npad // NS
    mesh = plsc.VectorSubcoreMesh(core_axis_name="c", subcore_axis_name="s")

    @functools.partial(
        pl.kernel,
        mesh=mesh,
        out_type=jax.ShapeDtypeStruct((NC, Npad, D), jnp.float32),
        scratch_types=[
            pltpu.VMEM((SCH,), jnp.int32),
            pltpu.VMEM((SCH,), jnp.int32),
            pltpu.VMEM((SCH, D), jnp.float32),
            pltpu.VMEM((SCH, D), jnp.float32),
            pltpu.VMEM_SHARED((Npad, D), jnp.float32),
            pltpu.SemaphoreType.DMA,
            pltpu.SemaphoreType.DMA,
            pltpu.SemaphoreType.DMA,
            pltpu.SemaphoreType.DMA,
        ],
    )
    def k(ef_hbm, ei_hbm, zero_hbm, out_hbm, idx0, idx1, ef0, ef1, acc_sh,
          si0, si1, se0, se1):
        c = lax.axis_index("c")
        s = lax.axis_index("s")
        zbase = s * rpt
        pltpu.sync_copy(zero_hbm.at[pl.ds(zbase, rpt)],
                        acc_sh.at[pl.ds(zbase, rpt)])
        plsc.subcore_barrier()

        base0 = c * epc + s * epw

        def start(i, idxb, efb, sib, seb):
            base = base0 + i * SCH
            pltpu.make_async_copy(ei_hbm.at[pl.ds(off + base, SCH)], idxb,
                                  sib).start()
            pltpu.make_async_copy(ef_hbm.at[pl.ds(base, SCH)], efb,
                                  seb).start()

        def flush(i, idxb, efb, sib, seb):
            base = base0 + i * SCH
            pltpu.make_async_copy(ei_hbm.at[pl.ds(off + base, SCH)], idxb,
                                  sib).wait()
            pltpu.make_async_copy(ef_hbm.at[pl.ds(base, SCH)], efb,
                                  seb).wait()
            pltpu.sync_copy(efb, acc_sh.at[idxb], add=True)

        start(0, idx0, ef0, si0, se0)

        @pl.loop(0, (nchunk - 1) // 2)
        def _(j):
            i0 = 2 * j
            start(i0 + 1, idx1, ef1, si1, se1)
            flush(i0, idx0, ef0, si0, se0)
            start(i0 + 2, idx0, ef0, si0, se0)
            flush(i0 + 1, idx1, ef1, si1, se1)

        flush(nchunk - 1, idx0, ef0, si0, se0)

        plsc.subcore_barrier()
        pltpu.sync_copy(acc_sh.at[pl.ds(zbase, rpt)],
                        out_hbm.at[c, pl.ds(zbase, rpt)])

    return k(ef, rowcol, zeros)


def _unpack_bf16_pair(w):
    u = lax.bitcast_convert_type(w, jnp.uint32)
    lo = lax.bitcast_convert_type((u & 0xFFFF).astype(jnp.uint16),
                                  jnp.bfloat16).astype(jnp.float32)
    hi = lax.bitcast_convert_type((u >> 16).astype(jnp.uint16),
                                  jnp.bfloat16).astype(jnp.float32)
    return jnp.concatenate([lo, hi], axis=1)


def _tc_edge_mlp(st, attr_t, off, W1e, b1e, W2e, b2e):
    E, D = st.shape
    D2 = D // 2
    DE = attr_t.shape[0]
    H = W2e.shape[0]
    BE = 6400
    blk_off = off // BE

    def body(st_ref, attr_ref, w1c_ref, b1_ref, w2_ref, b2_ref, out_ref):
        za = lax.dot_general(attr_ref[...], w1c_ref[...],
                             (((0,), (0,)), ((), ())),
                             preferred_element_type=jnp.float32)
        w = st_ref[...]
        z = (_unpack_bf16_pair(w[:, :D2]) + _unpack_bf16_pair(w[:, D2:])
             + za + b1_ref[...])
        m = z * jax.nn.sigmoid(z)
        y = jnp.dot(m, w2_ref[...],
                    preferred_element_type=jnp.float32) + b2_ref[...]
        out_ref[...] = y * jax.nn.sigmoid(y)

    return pl.pallas_call(
        body,
        grid=(E // BE,),
        in_specs=[
            pl.BlockSpec((BE, D), lambda i: (i, 0)),
            pl.BlockSpec((DE, BE), lambda i: (0, i + blk_off)),
            pl.BlockSpec((DE, H), lambda i: (0, 0)),
            pl.BlockSpec((1, H), lambda i: (0, 0)),
            pl.BlockSpec((H, H), lambda i: (0, 0)),
            pl.BlockSpec((1, H), lambda i: (0, 0)),
        ],
        out_specs=pl.BlockSpec((BE, H), lambda i: (i, 0)),
        out_shape=jax.ShapeDtypeStruct((E, H), jnp.float32),
    )(st, attr_t, W1e[2 * D:2 * D + DE], b1e.reshape(1, H), W2e,
      b2e.reshape(1, H))


def _tc_node_mlp(h, parts, W1n, b1n, W2n, b2n):
    N, D = h.shape
    H = parts[0].shape[1]
    DO = W2n.shape[1]
    BN = 2000

    def body(h_ref, p0_ref, p1_ref, p2_ref, p3_ref, w1_ref, b1_ref, w2_ref,
             b2_ref, out_ref):
        agg = ((p0_ref[...] + p1_ref[...]) + (p2_ref[...] + p3_ref[...]))
        w1 = w1_ref[...]
        z = (jnp.dot(h_ref[...], w1[0:D], preferred_element_type=jnp.float32)
             + jnp.dot(agg, w1[D:D + H], preferred_element_type=jnp.float32)
             + b1_ref[...])
        t = z * jax.nn.sigmoid(z)
        out_ref[...] = jnp.dot(t, w2_ref[...],
                               preferred_element_type=jnp.float32) + b2_ref[...]

    return pl.pallas_call(
        body,
        grid=(N // BN,),
        in_specs=[
            pl.BlockSpec((BN, D), lambda i: (i, 0)),
            pl.BlockSpec((BN, H), lambda i: (i, 0)),
            pl.BlockSpec((BN, H), lambda i: (i, 0)),
            pl.BlockSpec((BN, H), lambda i: (i, 0)),
            pl.BlockSpec((BN, H), lambda i: (i, 0)),
            pl.BlockSpec((D + H, H), lambda i: (0, 0)),
            pl.BlockSpec((1, H), lambda i: (0, 0)),
            pl.BlockSpec((H, DO), lambda i: (0, 0)),
            pl.BlockSpec((1, DO), lambda i: (0, 0)),
        ],
        out_specs=pl.BlockSpec((BN, DO), lambda i: (i, 0)),
        out_shape=jax.ShapeDtypeStruct((N, DO), jnp.float32),
    )(h, *parts, W1n, b1n.reshape(1, H), W2n, b2n.reshape(1, DO))


def kernel(h, edge_index, edge_attr, W1e, b1e, W2e, b2e, W1n, b1n, W2n, b2n):
    N = h.shape[0]
    E = edge_index.shape[1]
    Eh = E // NSPLIT
    attr_t = edge_attr.T
    rowcol = edge_index.reshape(2 * E)
    hs, ht = _tc_preproject(h, W1e)

    def pack(x):
        xb = x.astype(jnp.bfloat16)
        D = x.shape[1]
        lo = lax.bitcast_convert_type(xb[:, :D // 2], jnp.uint16).astype(jnp.uint32)
        hi = lax.bitcast_convert_type(xb[:, D // 2:], jnp.uint16).astype(jnp.uint32)
        return lax.bitcast_convert_type(lo | (hi << 16), jnp.int32)

    hs, ht = pack(hs), pack(ht)
    Npad = ((N + 8 * NS - 1) // (8 * NS)) * (8 * NS)
    zeros = jnp.zeros((Npad, h.shape[1]), jnp.float32)
    parts = []
    for ci in range(NSPLIT):
        lo = ci * Eh
        st = _sc_gather(hs, ht, rowcol, E, lo, Eh)
        ef = _tc_edge_mlp(st, attr_t, lo, W1e, b1e, W2e, b2e)
        p = _sc_segment_sum(ef, rowcol, lo, zeros)
        parts.extend([p[0, :N], p[1, :N]])
    return _tc_node_mlp(h, parts, W1n, b1n, W2n, b2n)


# speedup vs baseline: 2.0163x; 1.0048x over previous
import functools

import jax
import jax.numpy as jnp
from jax import lax
from jax.experimental import pallas as pl
from jax.experimental.pallas import tpu as pltpu
from jax.experimental.pallas import tpu_sc as plsc

NC, NS = 2, 16
NW = NC * NS
NSPLIT = 2


def _tc_preproject(h, W1e):
    N, D = h.shape
    H = W1e.shape[1]

    def body(h_ref, w1_ref, hs_ref, ht_ref):
        w1 = w1_ref[...]
        hv = h_ref[...]
        hs_ref[...] = jnp.dot(hv, w1[0:D], preferred_element_type=jnp.float32)
        ht_ref[...] = jnp.dot(hv, w1[D:2 * D], preferred_element_type=jnp.float32)

    return pl.pallas_call(
        body,
        grid=(1,),
        in_specs=[
            pl.BlockSpec((N, D), lambda i: (0, 0)),
            pl.BlockSpec(W1e.shape, lambda i: (0, 0)),
        ],
        out_specs=[
            pl.BlockSpec((N, H), lambda i: (0, 0)),
            pl.BlockSpec((N, H), lambda i: (0, 0)),
        ],
        out_shape=[jax.ShapeDtypeStruct((N, H), jnp.float32),
                   jax.ShapeDtypeStruct((N, H), jnp.float32)],
    )(h, W1e)


def _sc_gather(hs, ht, rowcol, E, off, Eh):
    N, D2 = hs.shape
    epw = Eh // NW
    CH = 200
    nchunk = epw // CH
    npair = nchunk // 2
    mesh = plsc.VectorSubcoreMesh(core_axis_name="c", subcore_axis_name="s")

    bufs = [pltpu.VMEM((CH,), jnp.int32), pltpu.VMEM((CH,), jnp.int32),
            pltpu.VMEM((CH, D2), jnp.int32), pltpu.VMEM((CH, D2), jnp.int32)]

    @functools.partial(
        pl.kernel,
        mesh=mesh,
        out_type=jax.ShapeDtypeStruct((Eh, 2 * D2), jnp.int32),
        scratch_types=bufs + bufs + [pltpu.SemaphoreType.DMA] * 12,
        compiler_params=pltpu.CompilerParams(use_tc_tiling_on_sc=False),
    )
    def k(hs_hbm, ht_hbm, ei_hbm, st_hbm,
          riA, ciA, srA, tgA, riB, ciB, srB, tgB, *sems):
        wid = lax.axis_index("s") * NC + lax.axis_index("c")
        base0 = wid * epw

        def idx_start(i, ri, ci, sms):
            base = off + base0 + i * CH
            return (pltpu.async_copy(ei_hbm.at[pl.ds(base, CH)], ri, sms[0]),
                    pltpu.async_copy(ei_hbm.at[pl.ds(E + base, CH)], ci,
                                     sms[1]))

        def gat_start(ri, ci, sr, tg, sms):
            return (pltpu.async_copy(hs_hbm.at[ri], sr, sms[2]),
                    pltpu.async_copy(ht_hbm.at[ci], tg, sms[3]))

        def out_start(i, sr, tg, sms):
            base = base0 + i * CH
            return (pltpu.async_copy(
                        sr, st_hbm.at[pl.ds(base, CH), pl.ds(0, D2)], sms[4]),
                    pltpu.async_copy(
                        tg, st_hbm.at[pl.ds(base, CH), pl.ds(D2, D2)], sms[5]))

        smA, smB = sems[:6], sems[6:]

        @pl.loop(0, npair)
        def _(j):
            i0 = 2 * j
            ia = idx_start(i0, riA, ciA, smA)
            ib = idx_start(i0 + 1, riB, ciB, smB)
            ia[0].wait()
            ia[1].wait()
            ga = gat_start(riA, ciA, srA, tgA, smA)
            ib[0].wait()
            ib[1].wait()
            ga[0].wait()
            ga[1].wait()
            wa = out_start(i0, srA, tgA, smA)
            gb = gat_start(riB, ciB, srB, tgB, smB)
            gb[0].wait()
            gb[1].wait()
            wb = out_start(i0 + 1, srB, tgB, smB)
            wa[0].wait()
            wa[1].wait()
            wb[0].wait()
            wb[1].wait()

        for i in range(2 * npair, nchunk):
            ia = idx_start(i, riA, ciA, smA)
            ia[0].wait()
            ia[1].wait()
            ga = gat_start(riA, ciA, srA, tgA, smA)
            ga[0].wait()
            ga[1].wait()
            wa = out_start(i, srA, tgA, smA)
            wa[0].wait()
            wa[1].wait()

    return k(hs, ht, rowcol)


def _sc_segment_sum(ef, rowcol, off, zeros):
    Eh, D = ef.shape
    Npad = zeros.shape[0]
    SCH = 40
    epc = Eh // NC
    epw = epc // NS
    nchunk = epw // SCH
    rpt = Npad // NS
    mesh = plsc.VectorSubcoreMesh(core_axis_name="c", subcore_axis_name="s")

    @functools.partial(
        pl.kernel,
        mesh=mesh,
        out_type=jax.ShapeDtypeStruct((NC, Npad, D), jnp.float32),
        scratch_types=[
            pltpu.VMEM((SCH,), jnp.int32),
            pltpu.VMEM((SCH,), jnp.int32),
            pltpu.VMEM((SCH, D), jnp.float32),
            pltpu.VMEM((SCH, D), jnp.float32),
            pltpu.VMEM_SHARED((Npad, D), jnp.float32),
            pltpu.SemaphoreType.DMA,
            pltpu.SemaphoreType.DMA,
            pltpu.SemaphoreType.DMA,
            pltpu.SemaphoreType.DMA,
        ],
    )
    def k(ef_hbm, ei_hbm, zero_hbm, out_hbm, idx0, idx1, ef0, ef1, acc_sh,
          si0, si1, se0, se1):
        c = lax.axis_index("c")
        s = lax.axis_index("s")
        zbase = s * rpt
        pltpu.sync_copy(zero_hbm.at[pl.ds(zbase, rpt)],
                        acc_sh.at[pl.ds(zbase, rpt)])
        plsc.subcore_barrier()

        base0 = c * epc + s * epw

        def start(i, idxb, efb, sib, seb):
            base = base0 + i * SCH
            pltpu.make_async_copy(ei_hbm.at[pl.ds(off + base, SCH)], idxb,
                                  sib).start()
            pltpu.make_async_copy(ef_hbm.at[pl.ds(base, SCH)], efb,
                                  seb).start()

        def flush(i, idxb, efb, sib, seb):
            base = base0 + i * SCH
            pltpu.make_async_copy(ei_hbm.at[pl.ds(off + base, SCH)], idxb,
                                  sib).wait()
            pltpu.make_async_copy(ef_hbm.at[pl.ds(base, SCH)], efb,
                                  seb).wait()
            pltpu.sync_copy(efb, acc_sh.at[idxb], add=True)

        start(0, idx0, ef0, si0, se0)

        @pl.loop(0, (nchunk - 1) // 2)
        def _(j):
            i0 = 2 * j
            start(i0 + 1, idx1, ef1, si1, se1)
            flush(i0, idx0, ef0, si0, se0)
            start(i0 + 2, idx0, ef0, si0, se0)
            flush(i0 + 1, idx1, ef1, si1, se1)

        flush(nchunk - 1, idx0, ef0, si0, se0)

        plsc.subcore_barrier()
        pltpu.sync_copy(acc_sh.at[pl.ds(zbase, rpt)],
                        out_hbm.at[c, pl.ds(zbase, rpt)])

    return k(ef, rowcol, zeros)


def _unpack_bf16_pair(w):
    u = lax.bitcast_convert_type(w, jnp.uint32)
    lo = lax.bitcast_convert_type((u & 0xFFFF).astype(jnp.uint16),
                                  jnp.bfloat16).astype(jnp.float32)
    hi = lax.bitcast_convert_type((u >> 16).astype(jnp.uint16),
                                  jnp.bfloat16).astype(jnp.float32)
    return jnp.concatenate([lo, hi], axis=1)


def _tc_edge_mlp(st, attr_t, off, W1e, b1e, W2e, b2e):
    E, D = st.shape
    D2 = D // 2
    DE = attr_t.shape[0]
    H = W2e.shape[0]
    BE = 6400
    blk_off = off // BE

    def body(st_ref, attr_ref, w1c_ref, b1_ref, w2_ref, b2_ref, out_ref):
        za = lax.dot_general(attr_ref[...], w1c_ref[...],
                             (((0,), (0,)), ((), ())),
                             preferred_element_type=jnp.float32)
        w = st_ref[...]
        z = (_unpack_bf16_pair(w[:, :D2]) + _unpack_bf16_pair(w[:, D2:])
             + za + b1_ref[...])
        m = z * jax.nn.sigmoid(z)
        y = jnp.dot(m, w2_ref[...],
                    preferred_element_type=jnp.float32) + b2_ref[...]
        out_ref[...] = y * jax.nn.sigmoid(y)

    return pl.pallas_call(
        body,
        grid=(E // BE,),
        in_specs=[
            pl.BlockSpec((BE, D), lambda i: (i, 0)),
            pl.BlockSpec((DE, BE), lambda i: (0, i + blk_off)),
            pl.BlockSpec((DE, H), lambda i: (0, 0)),
            pl.BlockSpec((1, H), lambda i: (0, 0)),
            pl.BlockSpec((H, H), lambda i: (0, 0)),
            pl.BlockSpec((1, H), lambda i: (0, 0)),
        ],
        out_specs=pl.BlockSpec((BE, H), lambda i: (i, 0)),
        out_shape=jax.ShapeDtypeStruct((E, H), jnp.float32),
    )(st, attr_t, W1e[2 * D:2 * D + DE], b1e.reshape(1, H), W2e,
      b2e.reshape(1, H))


def _tc_node_mlp(h, parts, W1n, b1n, W2n, b2n):
    N, D = h.shape
    H = parts[0].shape[1]
    DO = W2n.shape[1]
    BN = 2000

    def body(h_ref, p0_ref, p1_ref, p2_ref, p3_ref, w1_ref, b1_ref, w2_ref,
             b2_ref, out_ref):
        agg = ((p0_ref[...] + p1_ref[...]) + (p2_ref[...] + p3_ref[...]))
        w1 = w1_ref[...]
        z = (jnp.dot(h_ref[...], w1[0:D], preferred_element_type=jnp.float32)
             + jnp.dot(agg, w1[D:D + H], preferred_element_type=jnp.float32)
             + b1_ref[...])
        t = z * jax.nn.sigmoid(z)
        out_ref[...] = jnp.dot(t, w2_ref[...],
                               preferred_element_type=jnp.float32) + b2_ref[...]

    return pl.pallas_call(
        body,
        grid=(N // BN,),
        in_specs=[
            pl.BlockSpec((BN, D), lambda i: (i, 0)),
            pl.BlockSpec((BN, H), lambda i: (i, 0)),
            pl.BlockSpec((BN, H), lambda i: (i, 0)),
            pl.BlockSpec((BN, H), lambda i: (i, 0)),
            pl.BlockSpec((BN, H), lambda i: (i, 0)),
            pl.BlockSpec((D + H, H), lambda i: (0, 0)),
            pl.BlockSpec((1, H), lambda i: (0, 0)),
            pl.BlockSpec((H, DO), lambda i: (0, 0)),
            pl.BlockSpec((1, DO), lambda i: (0, 0)),
        ],
        out_specs=pl.BlockSpec((BN, DO), lambda i: (i, 0)),
        out_shape=jax.ShapeDtypeStruct((N, DO), jnp.float32),
    )(h, *parts, W1n, b1n.reshape(1, H), W2n, b2n.reshape(1, DO))


def kernel(h, edge_index, edge_attr, W1e, b1e, W2e, b2e, W1n, b1n, W2n, b2n):
    N = h.shape[0]
    E = edge_index.shape[1]
    Eh = E // NSPLIT
    attr_t = edge_attr.T
    rowcol = edge_index.reshape(2 * E)
    hs, ht = _tc_preproject(h, W1e)

    def pack(x):
        xb = x.astype(jnp.bfloat16)
        D = x.shape[1]
        lo = lax.bitcast_convert_type(xb[:, :D // 2], jnp.uint16).astype(jnp.uint32)
        hi = lax.bitcast_convert_type(xb[:, D // 2:], jnp.uint16).astype(jnp.uint32)
        return lax.bitcast_convert_type(lo | (hi << 16), jnp.int32)

    hs, ht = pack(hs), pack(ht)
    Npad = ((N + 8 * NS - 1) // (8 * NS)) * (8 * NS)
    zeros = jnp.zeros((Npad, h.shape[1]), jnp.float32)
    parts = []
    for ci in range(NSPLIT):
        lo = ci * Eh
        st = _sc_gather(hs, ht, rowcol, E, lo, Eh)
        ef = _tc_edge_mlp(st, attr_t, lo, W1e, b1e, W2e, b2e)
        p = _sc_segment_sum(ef, rowcol, lo, zeros)
        parts.extend([p[0, :N], p[1, :N]])
    return _tc_node_mlp(h, parts, W1n, b1n, W2n, b2n)
